# baseline (speedup 1.0000x reference)
import jax
import jax.numpy as jnp
from jax import lax
from jax.experimental import pallas as pl
from jax.experimental.pallas import tpu as pltpu
from jax.experimental.pallas import tpu_sc as plsc

VOCAB = 1000000
D = 64
PID_BASE = VOCAB - 3
NC, NS, L = 2, 16, 16
NW = NC * NS

B = 4096 * 200
C = 512
IDX_COLS = 128
IDX_ROWS = C // IDX_COLS
PER_W = B // NW
CHUNKS = PER_W // C


def _sc_body(idx_hbm, pp_hbm, table_hbm, out_hbm,
             idx_v, rows_v, pp_v, pos_v, off_v, sem):
    wid = lax.axis_index("s") * NC + lax.axis_index("c")
    row_base = wid * (PER_W // IDX_COLS)

    pltpu.sync_copy(pp_hbm, pp_v)

    def chunk_body(g, carry):
        r0 = row_base + g * IDX_ROWS
        pltpu.sync_copy(idx_hbm.at[pl.ds(r0, IDX_ROWS)], idx_v)

        copies = [
            pltpu.async_copy(table_hbm.at[idx_v.at[j]],
                             rows_v.at[pl.ds(j * IDX_COLS, IDX_COLS)], sem)
            for j in range(IDX_ROWS)
        ]
        for cp in copies:
            cp.wait()

        cnt = jnp.int32(0)
        for j in range(IDX_ROWS):
            for k in range(IDX_COLS // L):
                v = idx_v[j, pl.ds(k * L, L)]
                m = v >= PID_BASE
                loc = lax.iota(jnp.int32, L) + (j * IDX_COLS + k * L)
                plsc.store_compressed(pos_v.at[pl.ds(cnt, L)], loc, mask=m)
                plsc.store_compressed(off_v.at[pl.ds(cnt, L)], v - PID_BASE,
                                      mask=m)
                cnt = cnt + jnp.sum(m.astype(jnp.int32))

        def fix(i, c):
            p = pos_v[pl.ds(i, L)][0]
            o = off_v[pl.ds(i, L)][0]
            for kk in range(D // L):
                sl = pl.ds(kk * L, L)
                rows_v[p, sl] = pp_v[o, sl]
            return c

        lax.fori_loop(0, cnt, fix, 0)

        out_base = pl.multiple_of(wid * PER_W + g * C, C)
        pltpu.sync_copy(rows_v, out_hbm.at[pl.ds(out_base, C)])
        return carry

    lax.fori_loop(0, CHUNKS, chunk_body, 0)


@jax.jit
def _run(idx2d, prompt_params, table):
    mesh = plsc.VectorSubcoreMesh(core_axis_name="c", subcore_axis_name="s",
                                  num_cores=NC, num_subcores=NS)
    f = pl.kernel(
        _sc_body,
        out_type=jax.ShapeDtypeStruct((B, D), jnp.float32),
        mesh=mesh,
        scratch_types=[
            pltpu.VMEM((IDX_ROWS, IDX_COLS), jnp.int32),
            pltpu.VMEM((C, D), jnp.float32),
            pltpu.VMEM((4, D), jnp.float32),
            pltpu.VMEM((C + L,), jnp.int32),
            pltpu.VMEM((C + L,), jnp.int32),
            pltpu.SemaphoreType.DMA,
        ],
        compiler_params=pltpu.CompilerParams(needs_layout_passes=False,
                                             use_tc_tiling_on_sc=False),
    )
    return f(idx2d, prompt_params, table)


def kernel(input, table, prompt_params):
    idx2d = input.reshape(B // IDX_COLS, IDX_COLS).astype(jnp.int32)
    pp = jnp.concatenate(
        [prompt_params.astype(jnp.float32),
         jnp.zeros((1, D), jnp.float32)], axis=0)
    out = _run(idx2d, pp, table)
    return out.reshape(input.shape[0], input.shape[1], D)
